# baseline (speedup 1.0000x reference)
import functools
import math

import jax
import jax.numpy as jnp
import numpy as np
from jax import lax
from jax.experimental import pallas as pl
from jax.experimental.pallas import tpu as pltpu

EEG_CH = 14
C_OUT = 16
KH, KW = EEG_CH, 3
PAD_H = (KH - 1 + 1) // 2
NEG_SLOPE = 0.1
BN_EPS = 1e-5
ADAPT = 9
HC = EEG_CH + 2 * PAD_H - KH + 1
HP = HC // 2
ROWS = 2 * HP * C_OUT
KSUB = 48


def _adaptive_pool_matrix(in_size, out_size):
    m = np.zeros((out_size, in_size), dtype=np.float32)
    for i in range(out_size):
        s = (i * in_size) // out_size
        e = -((-(i + 1) * in_size) // out_size)
        m[i, s:e] = 1.0 / float(e - s)
    return m


def _fused_kernel(x_ref, w2_ref, g_ref, sh_ref, o_ref, x_scr, *, w_in, bn_blk):
    L = bn_blk * w_in
    lane = jax.lax.broadcasted_iota(jnp.int32, (16, w_in), 1)
    zero = jnp.zeros((), jnp.bfloat16)
    for s in range(bn_blk):
        xs = x_ref[s]
        sl = slice(s * w_in, (s + 1) * w_in)
        x_scr[0:16, sl] = jnp.where(lane == 0, zero,
                                    pltpu.roll(xs, 1, 1))
        x_scr[16:32, sl] = xs
        x_scr[32:48, sl] = jnp.where(lane == w_in - 1, zero,
                                     pltpu.roll(xs, w_in - 1, 1))
    a = x_scr[...]
    z = jnp.dot(w2_ref[...], a, preferred_element_type=jnp.float32)
    p = jnp.maximum(z[:ROWS // 2, :], z[ROWS // 2:, :])
    pm = jnp.maximum(p, pltpu.roll(p, L - 1, 1))
    pm = pm + sh_ref[...]
    pm = jnp.maximum(pm, NEG_SLOPE * pm)
    v = jnp.sum(pm * g_ref[...], axis=0, keepdims=True)
    o_ref[...] = v.reshape(bn_blk, w_in)


def kernel(x, w_conv, b_conv, bn_gamma, bn_beta, bn_mean, bn_var, w_fc, b_fc):
    n, cin, h_in, w_in = x.shape
    assert cin == 1 and h_in == EEG_CH and w_in % 2 == 0
    wp = w_in // 2

    scale = bn_gamma * lax.rsqrt(bn_var + BN_EPS)
    wk = w_conv[:, 0] * scale[:, None, None]
    shift_c = (b_conv - bn_mean) * scale + bn_beta

    h_arr = jnp.arange(2 * HP)
    r_arr = jnp.arange(KH)
    kh = r_arr[None, :] + PAD_H - h_arr[:, None]
    valid = ((kh >= 0) & (kh < KH)).astype(jnp.float32)
    w_hcrk = wk[:, jnp.clip(kh, 0, KH - 1), :] * valid[None, :, :, None]
    w_t = w_hcrk.transpose(1, 0, 3, 2)
    w_t = jnp.pad(w_t, ((0, 0), (0, 0), (0, 0), (0, 2)))
    w2 = (w_t.reshape(HP, 2, C_OUT, KSUB)
          .transpose(1, 0, 2, 3).reshape(ROWS, KSUB))

    ph = jnp.asarray(_adaptive_pool_matrix(HP, ADAPT))
    pw = jnp.asarray(_adaptive_pool_matrix(wp, ADAPT))
    wfc = w_fc.reshape(C_OUT, ADAPT, ADAPT)
    g = jnp.einsum('cij,ih,jw->chw', wfc, ph, pw,
                   precision=lax.Precision.HIGHEST)
    g_rows = g.transpose(1, 0, 2).reshape(HP * C_OUT, wp)
    g_full = jnp.zeros((HP * C_OUT, w_in), jnp.float32).at[:, ::2].set(g_rows)

    bn_blk = 1
    for cand in (64, 32, 16, 8, 4, 2):
        if n % cand == 0:
            bn_blk = cand
            break
    L = bn_blk * w_in
    nb = n // bn_blk
    g_tiled = jnp.tile(g_full, (1, bn_blk))

    w2 = w2.astype(jnp.bfloat16)

    xbf = x.reshape(n, EEG_CH, w_in).astype(jnp.bfloat16)
    xbf = jnp.concatenate(
        [xbf, jnp.ones((n, 1, w_in), jnp.bfloat16),
         jnp.zeros((n, 1, w_in), jnp.bfloat16)], axis=1)

    out = pl.pallas_call(
        functools.partial(_fused_kernel, w_in=w_in, bn_blk=bn_blk),
        out_shape=jax.ShapeDtypeStruct((n, w_in), jnp.float32),
        grid=(nb,),
        in_specs=[
            pl.BlockSpec((bn_blk, 16, w_in), lambda i: (i, 0, 0)),
            pl.BlockSpec((ROWS, KSUB), lambda i: (0, 0)),
            pl.BlockSpec((HP * C_OUT, L), lambda i: (0, 0)),
            pl.BlockSpec((HP * C_OUT, 1), lambda i: (0, 0)),
        ],
        out_specs=pl.BlockSpec((bn_blk, w_in), lambda i: (i, 0)),
        scratch_shapes=[pltpu.VMEM((KSUB, L), jnp.bfloat16)],
        compiler_params=pltpu.CompilerParams(
            dimension_semantics=("parallel",)),
    )(xbf, w2, g_tiled, jnp.tile(shift_c, HP).reshape(HP * C_OUT, 1))

    return out.sum(axis=1)[:, None] + b_fc[None, :]


# speedup vs baseline: 1.1350x; 1.1350x over previous
import functools
import math

import jax
import jax.numpy as jnp
import numpy as np
from jax import lax
from jax.experimental import pallas as pl
from jax.experimental.pallas import tpu as pltpu

EEG_CH = 14
C_OUT = 16
KH, KW = EEG_CH, 3
PAD_H = (KH - 1 + 1) // 2
NEG_SLOPE = 0.1
BN_EPS = 1e-5
ADAPT = 9
HC = EEG_CH + 2 * PAD_H - KH + 1
HP = HC // 2
ROWS = 2 * HP * C_OUT
KSUB = 48


def _adaptive_pool_matrix(in_size, out_size):
    m = np.zeros((out_size, in_size), dtype=np.float32)
    for i in range(out_size):
        s = (i * in_size) // out_size
        e = -((-(i + 1) * in_size) // out_size)
        m[i, s:e] = 1.0 / float(e - s)
    return m


def _fused_kernel(x_ref, w2_ref, g_ref, o_ref, x_scr, *, w_in, bn_blk):
    L = bn_blk * w_in
    lane = jax.lax.broadcasted_iota(jnp.int32, (16, w_in), 1)
    zero = jnp.zeros((), jnp.bfloat16)
    for s in range(bn_blk):
        xs = x_ref[s]
        sl = slice(s * w_in, (s + 1) * w_in)
        x_scr[0:16, sl] = jnp.where(lane == 0, zero,
                                    pltpu.roll(xs, 1, 1))
        x_scr[16:32, sl] = xs
        x_scr[32:48, sl] = jnp.where(lane == w_in - 1, zero,
                                     pltpu.roll(xs, w_in - 1, 1))
    a = x_scr[...]
    z = jnp.dot(w2_ref[...], a, preferred_element_type=jnp.float32)
    p = jnp.maximum(z[:ROWS // 2, :], z[ROWS // 2:, :])
    pm = jnp.maximum(p, pltpu.roll(p, L - 1, 1))
    pm = jnp.maximum(pm, NEG_SLOPE * pm)
    v = jnp.sum(pm * g_ref[...], axis=0, keepdims=True)
    o_ref[...] = v.reshape(bn_blk, w_in)


def kernel(x, w_conv, b_conv, bn_gamma, bn_beta, bn_mean, bn_var, w_fc, b_fc):
    n, cin, h_in, w_in = x.shape
    assert cin == 1 and h_in == EEG_CH and w_in % 2 == 0
    wp = w_in // 2

    scale = bn_gamma * lax.rsqrt(bn_var + BN_EPS)
    wk = w_conv[:, 0] * scale[:, None, None]
    shift_c = (b_conv - bn_mean) * scale + bn_beta

    h_arr = jnp.arange(2 * HP)
    r_arr = jnp.arange(KH)
    kh = r_arr[None, :] + PAD_H - h_arr[:, None]
    valid = ((kh >= 0) & (kh < KH)).astype(jnp.float32)
    w_hcrk = wk[:, jnp.clip(kh, 0, KH - 1), :] * valid[None, :, :, None]
    w_t = w_hcrk.transpose(1, 0, 3, 2)
    w_t = jnp.pad(w_t, ((0, 0), (0, 0), (0, 0), (0, 2)))
    sh_hi = shift_c.astype(jnp.bfloat16).astype(jnp.float32)
    w_t = w_t.at[:, :, 1, 14].set(sh_hi[None, :])
    w_t = w_t.at[:, :, 1, 15].set((shift_c - sh_hi)[None, :])
    w2 = (w_t.reshape(HP, 2, C_OUT, KSUB)
          .transpose(1, 0, 2, 3).reshape(ROWS, KSUB))

    ph = jnp.asarray(_adaptive_pool_matrix(HP, ADAPT))
    pw = jnp.asarray(_adaptive_pool_matrix(wp, ADAPT))
    wfc = w_fc.reshape(C_OUT, ADAPT, ADAPT)
    g = jnp.einsum('cij,ih,jw->chw', wfc, ph, pw,
                   precision=lax.Precision.HIGHEST)
    g_rows = g.transpose(1, 0, 2).reshape(HP * C_OUT, wp)
    g_full = jnp.zeros((HP * C_OUT, w_in), jnp.float32).at[:, ::2].set(g_rows)

    bn_blk = 1
    for cand in (64, 32, 16, 8, 4, 2):
        if n % cand == 0:
            bn_blk = cand
            break
    L = bn_blk * w_in
    nb = n // bn_blk
    g_tiled = jnp.tile(g_full, (1, bn_blk))

    w2 = w2.astype(jnp.bfloat16)

    xbf = x.reshape(n, EEG_CH, w_in).astype(jnp.bfloat16)
    xbf = jnp.concatenate(
        [xbf, jnp.ones((n, 2, w_in), jnp.bfloat16)], axis=1)

    out = pl.pallas_call(
        functools.partial(_fused_kernel, w_in=w_in, bn_blk=bn_blk),
        out_shape=jax.ShapeDtypeStruct((n, w_in), jnp.float32),
        grid=(nb,),
        in_specs=[
            pl.BlockSpec((bn_blk, 16, w_in), lambda i: (i, 0, 0)),
            pl.BlockSpec((ROWS, KSUB), lambda i: (0, 0)),
            pl.BlockSpec((HP * C_OUT, L), lambda i: (0, 0)),
        ],
        out_specs=pl.BlockSpec((bn_blk, w_in), lambda i: (i, 0)),
        scratch_shapes=[pltpu.VMEM((KSUB, L), jnp.bfloat16)],
        compiler_params=pltpu.CompilerParams(
            dimension_semantics=("parallel",)),
    )(xbf, w2, g_tiled)

    return out.sum(axis=1)[:, None] + b_fc[None, :]
